# baseline (speedup 1.0000x reference)
import jax
import jax.numpy as jnp
from jax import lax
from jax.experimental import pallas as pl
from jax.experimental.pallas import tpu as pltpu
from jax.experimental.pallas import tpu_sc as plsc

BATCH = 4096
SEQ = 200
D = 64
NC = 2
NS = 16
NW = NC * NS
ROWS = BATCH * SEQ
RPW = ROWS // NW
BPW = BATCH // NW
CHUNKS = RPW // SEQ
G1 = 104
G2 = SEQ - G1
LANES = 16


def _sc_body(idx_h, table_h, pe_h, out_h, idx_v, pe_v, buf0, buf1,
             sem0, sem1):
    wid = lax.axis_index("s") * NC + lax.axis_index("c")
    rbase = wid * RPW
    bbase = wid * BPW

    pltpu.sync_copy(idx_h.at[pl.ds(rbase, RPW)], idx_v)
    pltpu.sync_copy(pe_h, pe_v)

    def issue(c, buf, sem):
        row0 = c * SEQ
        h1 = pltpu.async_copy(
            table_h.at[idx_v.at[pl.ds(row0, G1)]],
            buf.at[pl.ds(0, G1)], sem)
        h2 = pltpu.async_copy(
            table_h.at[idx_v.at[pl.ds(row0 + G1, G2)]],
            buf.at[pl.ds(G1, G2)], sem)
        return h1, h2

    def consume(c, buf, handles):
        handles[0].wait()
        handles[1].wait()

        @pl.loop(0, SEQ, unroll=8)
        def _row(r):
            for j in range(D // LANES):
                sl = pl.ds(j * LANES, LANES)
                buf[r, sl] = buf[r, sl] + pe_v[r, sl]

        pltpu.sync_copy(buf, out_h.at[bbase + c])

    @pl.loop(0, CHUNKS // 2)
    def _group(g):
        c0 = 2 * g
        ha = issue(c0, buf0, sem0)
        hb = issue(c0 + 1, buf1, sem1)
        consume(c0, buf0, ha)
        consume(c0 + 1, buf1, hb)


@jax.jit
def _run(x_flat, table, pe_seq):
    mesh = plsc.VectorSubcoreMesh(
        core_axis_name="c", subcore_axis_name="s", num_cores=NC,
        num_subcores=NS)
    grid_kernel = pl.kernel(
        _sc_body,
        out_type=jax.ShapeDtypeStruct((BATCH, SEQ, D), jnp.float32),
        mesh=mesh,
        scratch_types=[
            pltpu.VMEM((RPW,), jnp.int32),
            pltpu.VMEM((SEQ, D), jnp.float32),
            pltpu.VMEM((SEQ, D), jnp.float32),
            pltpu.VMEM((SEQ, D), jnp.float32),
            pltpu.SemaphoreType.DMA,
            pltpu.SemaphoreType.DMA,
        ],
        compiler_params=pltpu.CompilerParams(use_tc_tiling_on_sc=False),
    )
    return grid_kernel(x_flat, table, pe_seq)


def kernel(x, table, pe):
    x_flat = x.reshape(ROWS)
    return _run(x_flat, table, pe[:SEQ])


# speedup vs baseline: 1.2495x; 1.2495x over previous
import jax
import jax.numpy as jnp
from jax import lax
from jax.experimental import pallas as pl
from jax.experimental.pallas import tpu as pltpu
from jax.experimental.pallas import tpu_sc as plsc

BATCH = 4096
SEQ = 200
D = 64
NC = 2
NS = 16
NW = NC * NS
ROWS = BATCH * SEQ
RPW = ROWS // NW
BPW = BATCH // NW
CHUNKS = RPW // SEQ
G1 = 104
G2 = SEQ - G1
LANES = 16


def _sc_body(idx_h, table_h, pe_h, out_h, idx_v, pe_v, buf0, buf1,
             sem0, sem1):
    wid = lax.axis_index("s") * NC + lax.axis_index("c")
    rbase = wid * RPW
    bbase = wid * BPW

    pltpu.sync_copy(idx_h.at[pl.ds(rbase, RPW)], idx_v)
    pltpu.sync_copy(pe_h, pe_v)

    def issue(c, buf, sem):
        row0 = c * SEQ
        h1 = pltpu.async_copy(
            table_h.at[idx_v.at[pl.ds(row0, G1)]],
            buf.at[pl.ds(0, G1)], sem)
        h2 = pltpu.async_copy(
            table_h.at[idx_v.at[pl.ds(row0 + G1, G2)]],
            buf.at[pl.ds(G1, G2)], sem)
        return h1, h2

    def consume(c, buf, handles):
        handles[0].wait()
        handles[1].wait()

        @pl.loop(0, SEQ)
        def _row(r):
            for j in range(D // LANES):
                sl = pl.ds(j * LANES, LANES)
                buf[r, sl] = buf[r, sl] + pe_v[r, sl]

        pltpu.sync_copy(buf, out_h.at[bbase + c])

    @pl.loop(0, CHUNKS // 2)
    def _group(g):
        c0 = 2 * g
        ha = issue(c0, buf0, sem0)
        hb = issue(c0 + 1, buf1, sem1)
        consume(c0, buf0, ha)
        consume(c0 + 1, buf1, hb)


@jax.jit
def _run(x_flat, table, pe_seq):
    mesh = plsc.VectorSubcoreMesh(
        core_axis_name="c", subcore_axis_name="s", num_cores=NC,
        num_subcores=NS)
    grid_kernel = pl.kernel(
        _sc_body,
        out_type=jax.ShapeDtypeStruct((BATCH, SEQ, D), jnp.float32),
        mesh=mesh,
        scratch_types=[
            pltpu.VMEM((RPW,), jnp.int32),
            pltpu.VMEM((SEQ, D), jnp.float32),
            pltpu.VMEM((SEQ, D), jnp.float32),
            pltpu.VMEM((SEQ, D), jnp.float32),
            pltpu.SemaphoreType.DMA,
            pltpu.SemaphoreType.DMA,
        ],
        compiler_params=pltpu.CompilerParams(use_tc_tiling_on_sc=False),
    )
    return grid_kernel(x_flat, table, pe_seq)


def kernel(x, table, pe):
    x_flat = x.reshape(ROWS)
    return _run(x_flat, table, pe[:SEQ])


# speedup vs baseline: 1.3091x; 1.0477x over previous
import jax
import jax.numpy as jnp
from jax import lax
from jax.experimental import pallas as pl
from jax.experimental.pallas import tpu as pltpu
from jax.experimental.pallas import tpu_sc as plsc

BATCH = 4096
SEQ = 200
D = 64
NC = 2
NS = 16
NW = NC * NS
ROWS = BATCH * SEQ
RPW = ROWS // NW
BPW = BATCH // NW
CHUNKS = RPW // SEQ
G1 = 104
G2 = SEQ - G1
LANES = 16
GRP = 4


def _sc_body(idx_h, table_h, pe_h, out_h, idx_v, pe_v,
             buf0, buf1, buf2, buf3, sem0, sem1, sem2, sem3,
             wsem0, wsem1, wsem2, wsem3):
    wid = lax.axis_index("s") * NC + lax.axis_index("c")
    rbase = wid * RPW
    bbase = wid * BPW

    pltpu.sync_copy(idx_h.at[pl.ds(rbase, RPW)], idx_v)
    pltpu.sync_copy(pe_h, pe_v)

    def issue(c, buf, sem):
        row0 = c * SEQ
        h1 = pltpu.async_copy(
            table_h.at[idx_v.at[pl.ds(row0, G1)]],
            buf.at[pl.ds(0, G1)], sem)
        h2 = pltpu.async_copy(
            table_h.at[idx_v.at[pl.ds(row0 + G1, G2)]],
            buf.at[pl.ds(G1, G2)], sem)
        return h1, h2

    def consume(c, buf, handles, wsem):
        handles[0].wait()
        handles[1].wait()

        @pl.loop(0, SEQ)
        def _row(r):
            for j in range(D // LANES):
                sl = pl.ds(j * LANES, LANES)
                buf[r, sl] = buf[r, sl] + pe_v[r, sl]

        return pltpu.async_copy(buf, out_h.at[bbase + c], wsem)

    bufs = (buf0, buf1, buf2, buf3)
    gsems = (sem0, sem1, sem2, sem3)
    wsems = (wsem0, wsem1, wsem2, wsem3)

    @pl.loop(0, CHUNKS // GRP)
    def _group(g):
        c0 = GRP * g
        ghs = [issue(c0 + k, bufs[k], gsems[k]) for k in range(GRP)]
        whs = [consume(c0 + k, bufs[k], ghs[k], wsems[k])
               for k in range(GRP)]
        for wh in whs:
            wh.wait()


@jax.jit
def _run(x_flat, table, pe_seq):
    mesh = plsc.VectorSubcoreMesh(
        core_axis_name="c", subcore_axis_name="s", num_cores=NC,
        num_subcores=NS)
    grid_kernel = pl.kernel(
        _sc_body,
        out_type=jax.ShapeDtypeStruct((BATCH, SEQ, D), jnp.float32),
        mesh=mesh,
        scratch_types=(
            [pltpu.VMEM((RPW,), jnp.int32),
             pltpu.VMEM((SEQ, D), jnp.float32)]
            + [pltpu.VMEM((SEQ, D), jnp.float32) for _ in range(GRP)]
            + [pltpu.SemaphoreType.DMA for _ in range(2 * GRP)]
        ),
        compiler_params=pltpu.CompilerParams(use_tc_tiling_on_sc=False),
    )
    return grid_kernel(x_flat, table, pe_seq)


def kernel(x, table, pe):
    x_flat = x.reshape(ROWS)
    return _run(x_flat, table, pe[:SEQ])


# speedup vs baseline: 1.3259x; 1.0128x over previous
import jax
import jax.numpy as jnp
from jax import lax
from jax.experimental import pallas as pl
from jax.experimental.pallas import tpu as pltpu
from jax.experimental.pallas import tpu_sc as plsc

BATCH = 4096
SEQ = 200
D = 64
NC = 2
NS = 16
NW = NC * NS
ROWS = BATCH * SEQ
RPW = ROWS // NW
BPW = BATCH // NW
CHUNKS = RPW // SEQ
G1 = 104
G2 = SEQ - G1
LANES = 16
GRP = 8


def _sc_body(idx_h, table_h, pe_h, out_h, idx_v, pe_v, *bufs_sems):
    wid = lax.axis_index("s") * NC + lax.axis_index("c")
    rbase = wid * RPW
    bbase = wid * BPW

    pltpu.sync_copy(pe_h, pe_v)

    def issue(k, buf, sem):
        row0 = k * SEQ
        h1 = pltpu.async_copy(
            table_h.at[idx_v.at[pl.ds(row0, G1)]],
            buf.at[pl.ds(0, G1)], sem)
        h2 = pltpu.async_copy(
            table_h.at[idx_v.at[pl.ds(row0 + G1, G2)]],
            buf.at[pl.ds(G1, G2)], sem)
        return h1, h2

    def consume(c, buf, handles, wsem):
        handles[0].wait()
        handles[1].wait()

        @pl.loop(0, SEQ)
        def _row(r):
            for j in range(D // LANES):
                sl = pl.ds(j * LANES, LANES)
                buf[r, sl] = buf[r, sl] + pe_v[r, sl]

        return pltpu.async_copy(buf, out_h.at[bbase + c], wsem)

    bufs = bufs_sems[0:GRP]
    gsems = bufs_sems[GRP:2 * GRP]
    wsems = bufs_sems[2 * GRP:3 * GRP]

    @pl.loop(0, CHUNKS // GRP)
    def _group(g):
        c0 = GRP * g
        pltpu.sync_copy(idx_h.at[pl.ds(rbase + c0 * SEQ, GRP * SEQ)],
                        idx_v)
        ghs = [issue(k, bufs[k], gsems[k]) for k in range(GRP)]
        whs = [consume(c0 + k, bufs[k], ghs[k], wsems[k])
               for k in range(GRP)]
        for wh in whs:
            wh.wait()


@jax.jit
def _run(x_flat, table, pe_seq):
    mesh = plsc.VectorSubcoreMesh(
        core_axis_name="c", subcore_axis_name="s", num_cores=NC,
        num_subcores=NS)
    grid_kernel = pl.kernel(
        _sc_body,
        out_type=jax.ShapeDtypeStruct((BATCH, SEQ, D), jnp.float32),
        mesh=mesh,
        scratch_types=(
            [pltpu.VMEM((GRP * SEQ,), jnp.int32),
             pltpu.VMEM((SEQ, D), jnp.float32)]
            + [pltpu.VMEM((SEQ, D), jnp.float32) for _ in range(GRP)]
            + [pltpu.SemaphoreType.DMA for _ in range(2 * GRP)]
        ),
        compiler_params=pltpu.CompilerParams(use_tc_tiling_on_sc=False),
    )
    return grid_kernel(x_flat, table, pe_seq)


def kernel(x, table, pe):
    x_flat = x.reshape(ROWS)
    return _run(x_flat, table, pe[:SEQ])
